# Initial kernel scaffold; baseline (speedup 1.0000x reference)
import jax
import jax.numpy as jnp
from jax.experimental import pallas as pl


def kernel(x, ptr, ratio, random_start):
    raise NotImplementedError("write your pallas kernel here")


# speedup vs baseline: 9.2697x; 9.2697x over previous
import functools
import math

import jax
import jax.numpy as jnp
import numpy as np
from jax import lax
from jax.experimental import pallas as pl
from jax.experimental.pallas import tpu as pltpu
from jax.experimental.pallas import tpu_sc as plsc

LANES = 16
UNROLL = 8


@functools.cache
def _fps_sc(B, L, k):
    mesh = plsc.VectorSubcoreMesh(core_axis_name="c", subcore_axis_name="s")
    num_chunks = L // LANES

    @functools.partial(
        pl.kernel,
        out_type=jax.ShapeDtypeStruct((B, k), jnp.int32),
        mesh=mesh,
        compiler_params=pltpu.CompilerParams(needs_layout_passes=False),
        scratch_types=[
            pltpu.VMEM((L,), jnp.float32),
            pltpu.VMEM((L,), jnp.float32),
            pltpu.VMEM((L,), jnp.float32),
            pltpu.VMEM((L,), jnp.float32),
            pltpu.VMEM((k,), jnp.int32),
            pltpu.VMEM((LANES,), jnp.int32),
        ],
    )
    def kern(x0_hbm, x1_hbm, x2_hbm, start_hbm, out_hbm, x0, x1, x2, md, sel, st):
        wid = lax.axis_index("s") * 2 + lax.axis_index("c")

        @pl.when(wid < B)
        def _():
            b = wid
            pltpu.sync_copy(x0_hbm.at[b], x0)
            pltpu.sync_copy(x1_hbm.at[b], x1)
            pltpu.sync_copy(x2_hbm.at[b], x2)
            pltpu.sync_copy(start_hbm, st)

            lanes = lax.iota(jnp.int32, LANES)
            inf16 = jnp.full((LANES,), jnp.inf, jnp.float32)

            def init_body(j, carry):
                md[pl.ds(j * LANES, LANES)] = inf16
                return carry

            lax.fori_loop(0, num_chunks, init_body, 0)

            def outer(i, cur):
                plsc.store_scatter(
                    sel, [jnp.full((LANES,), i, jnp.int32)], cur, mask=lanes == 0
                )
                c0 = plsc.load_gather(x0, [cur])
                c1 = plsc.load_gather(x1, [cur])
                c2 = plsc.load_gather(x2, [cur])

                def chunk(j, carry):
                    best, bidx = carry
                    for u in range(UNROLL):
                        off = (j * UNROLL + u) * LANES
                        t0 = x0[pl.ds(off, LANES)] - c0
                        t1 = x1[pl.ds(off, LANES)] - c1
                        t2 = x2[pl.ds(off, LANES)] - c2
                        d2 = t0 * t0 + t1 * t1 + t2 * t2
                        nmd = jnp.minimum(md[pl.ds(off, LANES)], d2)
                        md[pl.ds(off, LANES)] = nmd
                        upd = nmd > best
                        best = jnp.where(upd, nmd, best)
                        bidx = jnp.where(
                            upd, jnp.full((LANES,), j * UNROLL + u, jnp.int32), bidx
                        )
                    return best, bidx

                best0 = jnp.full((LANES,), -1.0, jnp.float32)
                bidx0 = jnp.zeros((LANES,), jnp.int32)
                best, bidx = lax.fori_loop(
                    0, num_chunks // UNROLL, chunk, (best0, bidx0)
                )
                m = jnp.max(best)
                gidx = bidx * LANES + lanes
                cand = jnp.where(best == m, gidx, jnp.full((LANES,), L, jnp.int32))
                nxt = jnp.min(cand)
                return jnp.full((LANES,), nxt, jnp.int32)

            cur0 = plsc.load_gather(st, [jnp.full((LANES,), b, jnp.int32)])
            lax.fori_loop(0, k, outer, cur0)
            pltpu.sync_copy(sel, out_hbm.at[b])

    return kern


def kernel(x, ptr, ratio, random_start):
    N, D = x.shape
    B = int(ptr.shape[0]) - 1
    L = N // B
    k = int(math.ceil(0.5 * L))
    xs = x.reshape(B, L, D)
    x0 = xs[:, :, 0]
    x1 = xs[:, :, 1]
    x2 = xs[:, :, 2]
    rng = np.random.RandomState(0)
    start_rand = jnp.asarray(rng.randint(0, L, size=(B,)), dtype=jnp.int32)
    start = jnp.where(
        jnp.asarray(random_start, dtype=bool),
        start_rand,
        jnp.zeros((B,), dtype=jnp.int32),
    )
    start_p = jnp.zeros((LANES,), jnp.int32).at[:B].set(start)
    sel = _fps_sc(B, L, k)(x0, x1, x2, start_p)
    flat = sel + ptr[:B].astype(jnp.int32)[:, None]
    return flat.reshape(-1)


# speedup vs baseline: 31.9834x; 3.4503x over previous
import functools
import math

import jax
import jax.numpy as jnp
import numpy as np
from jax import lax
from jax.experimental import pallas as pl
from jax.experimental.pallas import tpu as pltpu
from jax.experimental.pallas import tpu_sc as plsc

LANES = 16
UNROLL = 8


@functools.cache
def _fps_sc(B, L, k):
    mesh = plsc.VectorSubcoreMesh(core_axis_name="c", subcore_axis_name="s")
    num_chunks = L // LANES

    @functools.partial(
        pl.kernel,
        out_type=jax.ShapeDtypeStruct((B, k), jnp.int32),
        mesh=mesh,
        compiler_params=pltpu.CompilerParams(needs_layout_passes=False),
        scratch_types=[
            pltpu.VMEM((L,), jnp.float32),
            pltpu.VMEM((L,), jnp.float32),
            pltpu.VMEM((L,), jnp.float32),
            pltpu.VMEM((L,), jnp.float32),
            pltpu.VMEM((k,), jnp.int32),
            pltpu.VMEM((LANES,), jnp.int32),
        ],
    )
    def kern(x0_hbm, x1_hbm, x2_hbm, start_hbm, out_hbm, x0, x1, x2, md, sel, st):
        wid = lax.axis_index("s") * 2 + lax.axis_index("c")

        @pl.when(wid < B)
        def _():
            b = wid
            pltpu.sync_copy(x0_hbm.at[b], x0)
            pltpu.sync_copy(x1_hbm.at[b], x1)
            pltpu.sync_copy(x2_hbm.at[b], x2)
            pltpu.sync_copy(start_hbm, st)

            lanes = lax.iota(jnp.int32, LANES)
            inf16 = jnp.full((LANES,), jnp.inf, jnp.float32)

            @plsc.parallel_loop(0, num_chunks, unroll=UNROLL)
            def _init(j):
                md[pl.ds(j * LANES, LANES)] = inf16

            def outer(i, cur):
                plsc.store_scatter(
                    sel, [jnp.full((LANES,), i, jnp.int32)], cur, mask=lanes == 0
                )
                c0 = plsc.load_gather(x0, [cur])
                c1 = plsc.load_gather(x1, [cur])
                c2 = plsc.load_gather(x2, [cur])

                best0 = jnp.full((LANES,), -1.0, jnp.float32)
                bidx0 = jnp.zeros((LANES,), jnp.int32)

                @plsc.parallel_loop(
                    0, num_chunks, unroll=UNROLL, carry=(best0, bidx0)
                )
                def chunk(j, carry):
                    best, bidx = carry
                    off = j * LANES
                    t0 = x0[pl.ds(off, LANES)] - c0
                    t1 = x1[pl.ds(off, LANES)] - c1
                    t2 = x2[pl.ds(off, LANES)] - c2
                    d2 = t0 * t0 + t1 * t1 + t2 * t2
                    nmd = jnp.minimum(md[pl.ds(off, LANES)], d2)
                    md[pl.ds(off, LANES)] = nmd
                    upd = nmd > best
                    best = jnp.where(upd, nmd, best)
                    bidx = jnp.where(upd, jnp.full((LANES,), j, jnp.int32), bidx)
                    return best, bidx

                best, bidx = chunk
                m = jnp.max(best)
                gidx = bidx * LANES + lanes
                cand = jnp.where(best == m, gidx, jnp.full((LANES,), L, jnp.int32))
                nxt = jnp.min(cand)
                return jnp.full((LANES,), nxt, jnp.int32)

            cur0 = plsc.load_gather(st, [jnp.full((LANES,), b, jnp.int32)])
            lax.fori_loop(0, k, outer, cur0)
            pltpu.sync_copy(sel, out_hbm.at[b])

    return kern


def kernel(x, ptr, ratio, random_start):
    N, D = x.shape
    B = int(ptr.shape[0]) - 1
    L = N // B
    k = int(math.ceil(0.5 * L))
    xs = x.reshape(B, L, D)
    x0 = xs[:, :, 0]
    x1 = xs[:, :, 1]
    x2 = xs[:, :, 2]
    rng = np.random.RandomState(0)
    start_rand = jnp.asarray(rng.randint(0, L, size=(B,)), dtype=jnp.int32)
    start = jnp.where(
        jnp.asarray(random_start, dtype=bool),
        start_rand,
        jnp.zeros((B,), dtype=jnp.int32),
    )
    start_p = jnp.zeros((LANES,), jnp.int32).at[:B].set(start)
    sel = _fps_sc(B, L, k)(x0, x1, x2, start_p)
    flat = sel + ptr[:B].astype(jnp.int32)[:, None]
    return flat.reshape(-1)
